# baseline (speedup 1.0000x reference)
import jax
import jax.numpy as jnp
from jax.experimental import pallas as pl
from jax.experimental.pallas import tpu as pltpu

_ROWS = 1_000_000
_DIM = 64


def _copy_body(in_ref, out_ref, sem):
    cp = pltpu.make_async_copy(in_ref, out_ref, sem)
    cp.start()
    cp.wait()


def kernel(path_emb):
    return pl.pallas_call(
        _copy_body,
        in_specs=[pl.BlockSpec(memory_space=pl.ANY)],
        out_specs=pl.BlockSpec(memory_space=pl.ANY),
        out_shape=jax.ShapeDtypeStruct((_ROWS, _DIM), jnp.float32),
        scratch_shapes=[pltpu.SemaphoreType.DMA],
    )(path_emb)
